# baseline (speedup 1.0000x reference)
import jax
import jax.numpy as jnp
from jax import lax
from jax.experimental import pallas as pl
from jax.experimental.pallas import tpu as pltpu
from jax.experimental.pallas import tpu_sc as plsc

L = 16
NW = 32
NT = 4096 * 24
S = 128
CPW = NT // NW
GPW = CPW // L
SCHUNK = 4
NCH = S // SCHUNK
QSTG = 1024

_LUT20 = [0, 0, 1, 1, 2, 2, 3, 3, 3, 3, 4, 4, 4, 4, 5, 5, 6, 6, 7, 7] + [7] * 12


def _sc_body(qT, u2, tsp, iv, lut, out,
             qbuf, tab, tbuf, ibuf, lbuf,
             ub0, ub1, ob0, ob1, us0, us1, os0, os1):
    nc = 2
    wid = lax.axis_index("s") * nc + lax.axis_index("c")
    base = wid * CPW

    pltpu.sync_copy(tsp, tbuf)
    pltpu.sync_copy(iv, ibuf)
    pltpu.sync_copy(lut, lbuf)

    ivecs = [ibuf[j, :] for j in range(8)]
    tvecs = [tbuf[j, :] for j in range(8)]
    iota = lax.iota(jnp.int32, L)

    for p in range(CPW // QSTG):
        pltpu.sync_copy(qT.at[:, pl.ds(base + p * QSTG, QSTG)], qbuf)

        @pl.loop(0, QSTG // L)
        def _build(gg):
            sl = pl.ds(gg * L, L)
            gb = (p * (QSTG // L) + gg) * 256
            cum = qbuf[0, sl]
            for j in range(8):
                nxt = jnp.maximum(cum, qbuf[j + 1, sl])
                sj = (nxt - cum) * ivecs[j]
                tab[pl.ds(gb + j * L, L)] = cum - sj * tvecs[j]
                tab[pl.ds(gb + 128 + j * L, L)] = sj
                cum = nxt

    def uslice(c):
        return u2.at[pl.ds(c * SCHUNK, SCHUNK), pl.ds(base, CPW)]

    def oslice(c):
        return out.at[pl.ds(c * SCHUNK, SCHUNK), pl.ds(base, CPW)]

    pltpu.async_copy(uslice(0), ub0, us0)
    pltpu.async_copy(uslice(1), ub1, us1)

    twenty = jnp.full((L,), 20.0, dtype=jnp.float32)
    c128 = jnp.full((L,), 128, dtype=jnp.int32)

    @pl.loop(0, NCH, step=2)
    def _chunks(c0):
        for b, (ub, ob, us, osm) in enumerate(
            ((ub0, ob0, us0, os0), (ub1, ob1, us1, os1))):
            c = c0 + b
            pltpu.make_async_copy(uslice(c), ub, us).wait()

            @pl.when(c >= 2)
            def _():
                pltpu.make_async_copy(ob, oslice(c), osm).wait()

            @pl.loop(0, GPW)
            def _grp(g):
                sl = pl.ds(g * L, L)
                gv = iota + g * 256
                for r in range(SCHUNK):
                    uv = ub[r, sl]
                    k = (uv * twenty).astype(jnp.int32)
                    idx = plsc.load_gather(lbuf, [k])
                    addr_a = (idx << 4) + gv
                    av = plsc.load_gather(tab, [addr_a])
                    bv = plsc.load_gather(tab, [addr_a + c128])
                    ob[r, sl] = jnp.maximum(av + bv * uv, 0.0)

            @pl.when(c + 2 < NCH)
            def _():
                pltpu.async_copy(uslice(c + 2), ub, us)

            pltpu.async_copy(ob, oslice(c), osm)

    pltpu.make_async_copy(ob0, oslice(NCH - 2), os0).wait()
    pltpu.make_async_copy(ob1, oslice(NCH - 1), os1).wait()


@jax.jit
def kernel(q_curve, u, taus):
    qT = q_curve.reshape(NT, 9).T
    u2 = u.reshape(S, NT)
    dt = taus[1:] - taus[:-1]
    ivs = 1.0 / (dt + 1e-12)
    tsp = jnp.broadcast_to(taus[:8, None], (8, L)).astype(jnp.float32)
    ivb = jnp.broadcast_to(ivs[:, None], (8, L)).astype(jnp.float32)
    lut = jnp.asarray(_LUT20, dtype=jnp.int32)

    mesh = plsc.VectorSubcoreMesh(core_axis_name="c", subcore_axis_name="s")
    run = pl.kernel(
        _sc_body,
        out_type=jax.ShapeDtypeStruct((S, NT), jnp.float32),
        mesh=mesh,
        compiler_params=pltpu.CompilerParams(needs_layout_passes=False),
        scratch_types=[
            pltpu.VMEM((9, QSTG), jnp.float32),
            pltpu.VMEM((GPW * 256,), jnp.float32),
            pltpu.VMEM((8, L), jnp.float32),
            pltpu.VMEM((8, L), jnp.float32),
            pltpu.VMEM((32,), jnp.int32),
            pltpu.VMEM((SCHUNK, CPW), jnp.float32),
            pltpu.VMEM((SCHUNK, CPW), jnp.float32),
            pltpu.VMEM((SCHUNK, CPW), jnp.float32),
            pltpu.VMEM((SCHUNK, CPW), jnp.float32),
            pltpu.SemaphoreType.DMA,
            pltpu.SemaphoreType.DMA,
            pltpu.SemaphoreType.DMA,
            pltpu.SemaphoreType.DMA,
        ],
    )
    scen = run(qT, u2, tsp, ivb, lut)
    return scen.reshape(S, 4096, 24)


# speedup vs baseline: 1.5598x; 1.5598x over previous
import jax
import jax.numpy as jnp
from jax import lax
from jax.experimental import pallas as pl
from jax.experimental.pallas import tpu as pltpu
from jax.experimental.pallas import tpu_sc as plsc

L = 16
NW = 32
NT = 4096 * 24
S = 128
CPW = NT // NW
GPW = CPW // L
SCHUNK = 4
NCH = S // SCHUNK
QSTG = 1024


def _sc_body(qT, u2, tsp, iv, out,
             qbuf, coef, tbuf, ibuf,
             ub0, ub1, ob0, ob1, us0, us1, os0, os1):
    nc = 2
    wid = lax.axis_index("s") * nc + lax.axis_index("c")
    base = wid * CPW

    pltpu.sync_copy(tsp, tbuf)
    pltpu.sync_copy(iv, ibuf)

    ivecs = [ibuf[j, :] for j in range(8)]
    t0 = tbuf[0, :]
    tvecs = [tbuf[j, :] for j in range(1, 8)]

    for p in range(CPW // QSTG):
        pltpu.sync_copy(qT.at[:, pl.ds(base + p * QSTG, QSTG)], qbuf)

        @pl.loop(0, QSTG // L)
        def _build(gg):
            sl = pl.ds(gg * L, L)
            osl = pl.ds(p * QSTG + gg * L, L)
            cum = qbuf[0, sl]
            first = cum
            svecs = []
            for j in range(8):
                nxt = jnp.maximum(cum, qbuf[j + 1, sl])
                svecs.append((nxt - cum) * ivecs[j])
                cum = nxt
            coef[0, osl] = first - svecs[0] * t0
            coef[1, osl] = svecs[0]
            for j in range(1, 8):
                coef[1 + j, osl] = svecs[j] - svecs[j - 1]

    def uslice(c):
        return u2.at[pl.ds(c * SCHUNK, SCHUNK), pl.ds(base, CPW)]

    def oslice(c):
        return out.at[pl.ds(c * SCHUNK, SCHUNK), pl.ds(base, CPW)]

    pltpu.async_copy(uslice(0), ub0, us0)
    pltpu.async_copy(uslice(1), ub1, us1)

    @pl.loop(0, NCH, step=2)
    def _chunks(c0):
        for b, (ub, ob, us, osm) in enumerate(
            ((ub0, ob0, us0, os0), (ub1, ob1, us1, os1))):
            c = c0 + b
            pltpu.make_async_copy(uslice(c), ub, us).wait()

            @pl.when(c >= 2)
            def _():
                pltpu.make_async_copy(ob, oslice(c), osm).wait()

            @pl.loop(0, GPW)
            def _grp(g):
                sl = pl.ds(g * L, L)
                cvecs = [coef[j, sl] for j in range(9)]
                for r in range(SCHUNK):
                    uv = ub[r, sl]
                    terms = [cvecs[0] + cvecs[1] * uv]
                    for j in range(1, 8):
                        terms.append(
                            cvecs[1 + j] * jnp.maximum(uv - tvecs[j - 1], 0.0))
                    while len(terms) > 1:
                        terms = [terms[i] + terms[i + 1]
                                 for i in range(0, len(terms) - 1, 2)] + (
                                     [terms[-1]] if len(terms) % 2 else [])
                    ob[r, sl] = jnp.maximum(terms[0], 0.0)

            @pl.when(c + 2 < NCH)
            def _():
                pltpu.async_copy(uslice(c + 2), ub, us)

            pltpu.async_copy(ob, oslice(c), osm)

    pltpu.make_async_copy(ob0, oslice(NCH - 2), os0).wait()
    pltpu.make_async_copy(ob1, oslice(NCH - 1), os1).wait()


@jax.jit
def kernel(q_curve, u, taus):
    qT = q_curve.reshape(NT, 9).T
    u2 = u.reshape(S, NT)
    dt = taus[1:] - taus[:-1]
    ivs = 1.0 / (dt + 1e-12)
    tsp = jnp.broadcast_to(taus[:8, None], (8, L)).astype(jnp.float32)
    ivb = jnp.broadcast_to(ivs[:, None], (8, L)).astype(jnp.float32)

    mesh = plsc.VectorSubcoreMesh(core_axis_name="c", subcore_axis_name="s")
    run = pl.kernel(
        _sc_body,
        out_type=jax.ShapeDtypeStruct((S, NT), jnp.float32),
        mesh=mesh,
        compiler_params=pltpu.CompilerParams(needs_layout_passes=False),
        scratch_types=[
            pltpu.VMEM((9, QSTG), jnp.float32),
            pltpu.VMEM((9, CPW), jnp.float32),
            pltpu.VMEM((8, L), jnp.float32),
            pltpu.VMEM((8, L), jnp.float32),
            pltpu.VMEM((SCHUNK, CPW), jnp.float32),
            pltpu.VMEM((SCHUNK, CPW), jnp.float32),
            pltpu.VMEM((SCHUNK, CPW), jnp.float32),
            pltpu.VMEM((SCHUNK, CPW), jnp.float32),
            pltpu.SemaphoreType.DMA,
            pltpu.SemaphoreType.DMA,
            pltpu.SemaphoreType.DMA,
            pltpu.SemaphoreType.DMA,
        ],
    )
    scen = run(qT, u2, tsp, ivb)
    return scen.reshape(S, 4096, 24)
